# baseline (speedup 1.0000x reference)
import jax
import jax.numpy as jnp
from jax.experimental import pallas as pl
from jax.experimental.pallas import tpu as pltpu

_OC = 64
_HID = 512
_L = 32
_C = 20

_BLOCKS = (
    (0, 10, 0, 240),
    (10, 10, 180, 240),
    (20, 10, 380, 240),
    (30, 2, 500, 140),
)


def _fused_kernel(x_ref, wc_ref, bc_ref, wl_ref, bl_ref, o_ref,
                  wb0, wb1, wb2, wb3, bcs, wls):
    j = pl.program_id(1)

    @pl.when(j == 0)
    def _prep():
        w = wc_ref[...]
        w0t = jnp.transpose(w[:, :, 0])
        w1t = jnp.transpose(w[:, :, 1])
        for (ps, npos, off, kw), wb in zip(_BLOCKS, (wb0, wb1, wb2, wb3)):
            nt = kw // _C
            t = off // _C + jax.lax.broadcasted_iota(jnp.int32, (nt, npos + 1), 0)
            p = ps + jax.lax.broadcasted_iota(jnp.int32, (nt, npos + 1), 1)
            m1 = (t == p).astype(jnp.float32)
            m0 = (t + 1 == p).astype(jnp.float32)
            blk = (m1[:, None, :, None] * w1t[None, :, None, :]
                   + m0[:, None, :, None] * w0t[None, :, None, :])
            wb[...] = blk.reshape(kw, (npos + 1) * _OC).astype(jnp.bfloat16)
        bcs[...] = jnp.concatenate([bc_ref[...]] * _L, axis=1)
        vt = jnp.transpose(wl_ref[...])
        wls[...] = (vt.reshape(_OC, _L, _HID)
                    .transpose(1, 0, 2).reshape(_L * _OC, _HID)
                    .astype(jnp.bfloat16))

    xb = x_ref[...].astype(jnp.bfloat16)
    parts = []
    for (ps, npos, off, kw), wb in zip(_BLOCKS, (wb0, wb1, wb2, wb3)):
        conv = jnp.dot(xb[:, off:off + kw], wb[...],
                       preferred_element_type=jnp.float32)
        n = npos * _OC
        parts.append(jnp.maximum(conv[:, :n], conv[:, _OC:_OC + n]))
    pooled = (jnp.concatenate(parts, axis=1) + bcs[...]).astype(jnp.bfloat16)
    o_ref[...] = (jnp.dot(pooled, wls[...],
                          preferred_element_type=jnp.float32)
                  + bl_ref[...]).astype(o_ref.dtype)


def kernel(protein_ft, w_conv, b_conv, w_lin, b_lin):
    B, L, C = protein_ft.shape
    assert (L, C) == (_L, _C), (L, C)
    f32 = jnp.float32

    x2d = jnp.concatenate([protein_ft[:, t, :] for t in range(L)],
                          axis=1).astype(f32)
    TB = 512 if B >= 1024 else -(-B // 8) * 8
    B_pad = -(-B // TB) * TB
    if B_pad != B:
        x2d = jnp.pad(x2d, ((0, B_pad - B), (0, 0)))
    nbt = B_pad // TB
    ncores = 2 if nbt % 2 == 0 else 1
    nj = nbt // ncores

    wcf = w_conv.astype(f32)
    bcf = b_conv.astype(f32)[None, :]
    wlf = w_lin.astype(f32)
    blf = b_lin.astype(f32)[None, :]

    out = pl.pallas_call(
        _fused_kernel,
        out_shape=jax.ShapeDtypeStruct((B_pad, _HID), f32),
        grid=(ncores, nj),
        in_specs=[
            pl.BlockSpec((TB, L * C), lambda c, j, nj=nj: (c * nj + j, 0)),
            pl.BlockSpec(wcf.shape, lambda c, j: (0, 0, 0)),
            pl.BlockSpec((1, _OC), lambda c, j: (0, 0)),
            pl.BlockSpec((_HID, L * _OC), lambda c, j: (0, 0)),
            pl.BlockSpec((1, _HID), lambda c, j: (0, 0)),
        ],
        out_specs=pl.BlockSpec((TB, _HID), lambda c, j, nj=nj: (c * nj + j, 0)),
        scratch_shapes=[
            pltpu.VMEM((kw, (npos + 1) * _OC), jnp.bfloat16)
            for (ps, npos, off, kw) in _BLOCKS
        ] + [
            pltpu.VMEM((1, _L * _OC), f32),
            pltpu.VMEM((_L * _OC, _HID), jnp.bfloat16),
        ],
        compiler_params=pltpu.CompilerParams(
            dimension_semantics=("parallel", "arbitrary"),
            vmem_limit_bytes=64 << 20),
    )(x2d, wcf, bcf, wlf, blf)
    return out[:B]


# speedup vs baseline: 2.8623x; 2.8623x over previous
import jax
import jax.numpy as jnp
from jax.experimental import pallas as pl
from jax.experimental.pallas import tpu as pltpu

_OC = 64
_HID = 512
_L = 32
_C = 20


def _fused_kernel(x_ref, wc_ref, bc_ref, wl_ref, bl_ref, o_ref,
                  wband, bcs, wls, bls):
    j = pl.program_id(1)

    @pl.when(j == 0)
    def _prep():
        w = wc_ref[...]
        m1 = (jax.lax.broadcasted_iota(jnp.int32, (_L + 1, _L), 0)
              == jax.lax.broadcasted_iota(jnp.int32, (_L + 1, _L), 1))
        m0 = (jax.lax.broadcasted_iota(jnp.int32, (_L + 1, _L), 0)
              == jax.lax.broadcasted_iota(jnp.int32, (_L + 1, _L), 1) + 1)
        band = (m1.astype(jnp.float32)[:, None, None, :] * w[None, :, :, 1][..., None]
                + m0.astype(jnp.float32)[:, None, None, :] * w[None, :, :, 0][..., None])
        wband[...] = band.reshape((_L + 1) * _OC, _C * _L).astype(jnp.bfloat16)
        bcol = jnp.transpose(bc_ref[...])
        bcs[...] = jnp.concatenate([bcol] * _L, axis=0)
        wls[...] = (wl_ref[...].reshape(_HID, _OC, _L)
                    .transpose(0, 2, 1).reshape(_HID, _L * _OC)
                    .astype(jnp.bfloat16))
        bls[...] = jnp.transpose(bl_ref[...])

    xb = x_ref[...].astype(jnp.bfloat16)
    convT = jnp.dot(wband[...], xb,
                    preferred_element_type=jnp.float32)
    pooledT = (jnp.maximum(convT[:_L * _OC], convT[_OC:])
               + bcs[...]).astype(jnp.bfloat16)
    o_ref[...] = (jnp.dot(wls[...], pooledT,
                          preferred_element_type=jnp.float32)
                  + bls[...]).astype(o_ref.dtype)


def kernel(protein_ft, w_conv, b_conv, w_lin, b_lin):
    B, L, C = protein_ft.shape
    assert (L, C) == (_L, _C), (L, C)
    f32 = jnp.float32

    xt = protein_ft.transpose(2, 1, 0).reshape(C * L, B).astype(f32)
    TB = 512 if B >= 1024 else -(-B // 128) * 128
    B_pad = -(-B // TB) * TB
    if B_pad != B:
        xt = jnp.pad(xt, ((0, 0), (0, B_pad - B)))
    nbt = B_pad // TB
    ncores = 2 if nbt % 2 == 0 else 1
    nj = nbt // ncores

    wcf = w_conv.astype(f32)
    bcf = b_conv.astype(f32)[None, :]
    wlf = w_lin.astype(f32)
    blf = b_lin.astype(f32)[None, :]

    out = pl.pallas_call(
        _fused_kernel,
        out_shape=jax.ShapeDtypeStruct((_HID, B_pad), f32),
        grid=(ncores, nj),
        in_specs=[
            pl.BlockSpec((C * L, TB), lambda c, j, nj=nj: (0, c * nj + j)),
            pl.BlockSpec(wcf.shape, lambda c, j: (0, 0, 0)),
            pl.BlockSpec((1, _OC), lambda c, j: (0, 0)),
            pl.BlockSpec((_HID, L * _OC), lambda c, j: (0, 0)),
            pl.BlockSpec((1, _HID), lambda c, j: (0, 0)),
        ],
        out_specs=pl.BlockSpec((_HID, TB), lambda c, j, nj=nj: (0, c * nj + j)),
        scratch_shapes=[
            pltpu.VMEM(((_L + 1) * _OC, _C * _L), jnp.bfloat16),
            pltpu.VMEM((_L * _OC, 1), f32),
            pltpu.VMEM((_HID, _L * _OC), jnp.bfloat16),
            pltpu.VMEM((_HID, 1), f32),
        ],
        compiler_params=pltpu.CompilerParams(
            dimension_semantics=("parallel", "arbitrary"),
            vmem_limit_bytes=64 << 20),
    )(xt, wcf, bcf, wlf, blf)
    return jnp.transpose(out[:, :B])


# speedup vs baseline: 3.4061x; 1.1900x over previous
import jax
import jax.numpy as jnp
from jax.experimental import pallas as pl
from jax.experimental.pallas import tpu as pltpu

_OC = 64
_HID = 512
_L = 32
_C = 20


def _fused_kernel(x_ref, wc_ref, bc_ref, wl_ref, bl_ref, o_ref,
                  wband, bcs, wls, bls):
    j = pl.program_id(0)

    @pl.when(j == 0)
    def _prep():
        w = wc_ref[...]
        m1 = (jax.lax.broadcasted_iota(jnp.int32, (_L + 1, _L), 0)
              == jax.lax.broadcasted_iota(jnp.int32, (_L + 1, _L), 1))
        m0 = (jax.lax.broadcasted_iota(jnp.int32, (_L + 1, _L), 0)
              == jax.lax.broadcasted_iota(jnp.int32, (_L + 1, _L), 1) + 1)
        band = (m1.astype(jnp.float32)[:, None, None, :] * w[None, :, :, 1][..., None]
                + m0.astype(jnp.float32)[:, None, None, :] * w[None, :, :, 0][..., None])
        wband[...] = band.reshape((_L + 1) * _OC, _C * _L).astype(jnp.bfloat16)
        bcol = jnp.transpose(bc_ref[...])
        bcs[...] = jnp.concatenate([bcol] * _L, axis=0)
        wls[...] = (wl_ref[...].reshape(_HID, _OC, _L)
                    .transpose(0, 2, 1).reshape(_HID, _L * _OC)
                    .astype(jnp.bfloat16))
        bls[...] = jnp.transpose(bl_ref[...])

    xb = x_ref[...].astype(jnp.bfloat16)
    convT = jnp.dot(wband[...], xb,
                    preferred_element_type=jnp.float32)
    pooledT = (jnp.maximum(convT[:_L * _OC], convT[_OC:])
               + bcs[...]).astype(jnp.bfloat16)
    o_ref[...] = (jnp.dot(wls[...], pooledT,
                          preferred_element_type=jnp.float32)
                  + bls[...]).astype(o_ref.dtype)


def kernel(protein_ft, w_conv, b_conv, w_lin, b_lin):
    B, L, C = protein_ft.shape
    assert (L, C) == (_L, _C), (L, C)
    f32 = jnp.float32

    xt = protein_ft.transpose(2, 1, 0).reshape(C * L, B).astype(f32)
    TB = 1024 if B >= 1024 else -(-B // 128) * 128
    B_pad = -(-B // TB) * TB
    if B_pad != B:
        xt = jnp.pad(xt, ((0, 0), (0, B_pad - B)))
    nbt = B_pad // TB

    wcf = w_conv.astype(f32)
    bcf = b_conv.astype(f32)[None, :]
    wlf = w_lin.astype(f32)
    blf = b_lin.astype(f32)[None, :]

    out = pl.pallas_call(
        _fused_kernel,
        out_shape=jax.ShapeDtypeStruct((_HID, B_pad), f32),
        grid=(nbt,),
        in_specs=[
            pl.BlockSpec((C * L, TB), lambda j: (0, j)),
            pl.BlockSpec(wcf.shape, lambda j: (0, 0, 0)),
            pl.BlockSpec((1, _OC), lambda j: (0, 0)),
            pl.BlockSpec((_HID, L * _OC), lambda j: (0, 0)),
            pl.BlockSpec((1, _HID), lambda j: (0, 0)),
        ],
        out_specs=pl.BlockSpec((_HID, TB), lambda j: (0, j)),
        scratch_shapes=[
            pltpu.VMEM(((_L + 1) * _OC, _C * _L), jnp.bfloat16),
            pltpu.VMEM((_L * _OC, 1), f32),
            pltpu.VMEM((_HID, _L * _OC), jnp.bfloat16),
            pltpu.VMEM((_HID, 1), f32),
        ],
        compiler_params=pltpu.CompilerParams(
            dimension_semantics=("arbitrary",),
            vmem_limit_bytes=64 << 20),
    )(xt, wcf, bcf, wlf, blf)
    return jnp.transpose(out[:, :B])


# speedup vs baseline: 3.6479x; 1.0710x over previous
import jax
import jax.numpy as jnp
from jax.experimental import pallas as pl
from jax.experimental.pallas import tpu as pltpu

_OC = 64
_HID = 512
_L = 32
_C = 20


def _fused_kernel(x_ref, wc_ref, bc_ref, wl_ref, bl_ref, o_ref,
                  wband, bcs, wls, bls):
    j = pl.program_id(0)

    @pl.when(j == 0)
    def _prep():
        w = wc_ref[...]
        nr, nl = (_L + 1) * _OC, _C * _L
        w1e = jnp.repeat(w[:, :, 1], _L, axis=1)
        w0e = jnp.repeat(w[:, :, 0], _L, axis=1)
        vals1 = jnp.concatenate([w1e] * (_L + 1), axis=0)
        vals0 = jnp.concatenate([w0e] * (_L + 1), axis=0)
        t_of_lane = jax.lax.broadcasted_iota(jnp.int32, (nr, nl), 1) % _L
        p_of_row = jax.lax.broadcasted_iota(jnp.int32, (nr, nl), 0) // _OC
        zero = jnp.zeros((), jnp.float32)
        band = (jnp.where(t_of_lane == p_of_row, vals1, zero)
                + jnp.where(t_of_lane + 1 == p_of_row, vals0, zero))
        wband[...] = band.astype(jnp.bfloat16)
        wls[...] = (wl_ref[...].reshape(_HID, _OC, _L)
                    .transpose(0, 2, 1).reshape(_HID, _L * _OC)
                    .astype(jnp.bfloat16))
        bcol = jnp.transpose(bc_ref[...])
        bcs[...] = jnp.concatenate([bcol] * _L, axis=0)
        bls[...] = (jnp.dot(wls[...].astype(jnp.float32), bcs[...],
                            preferred_element_type=jnp.float32)
                    + jnp.transpose(bl_ref[...]))

    xb = x_ref[...].astype(jnp.bfloat16)
    convT = jnp.dot(wband[...], xb,
                    preferred_element_type=jnp.float32)
    convb = convT.astype(jnp.bfloat16)
    pooledT = jnp.maximum(convb[:_L * _OC], convb[_OC:])
    o_ref[...] = (jnp.dot(wls[...], pooledT,
                          preferred_element_type=jnp.float32)
                  + bls[...]).astype(o_ref.dtype)


def kernel(protein_ft, w_conv, b_conv, w_lin, b_lin):
    B, L, C = protein_ft.shape
    assert (L, C) == (_L, _C), (L, C)
    f32 = jnp.float32

    xt = protein_ft.transpose(2, 1, 0).reshape(C * L, B).astype(f32)
    TB = 1024 if B >= 1024 else -(-B // 128) * 128
    B_pad = -(-B // TB) * TB
    if B_pad != B:
        xt = jnp.pad(xt, ((0, 0), (0, B_pad - B)))
    nbt = B_pad // TB

    wcf = w_conv.astype(f32)
    bcf = b_conv.astype(f32)[None, :]
    wlf = w_lin.astype(f32)
    blf = b_lin.astype(f32)[None, :]

    out = pl.pallas_call(
        _fused_kernel,
        out_shape=jax.ShapeDtypeStruct((_HID, B_pad), f32),
        grid=(nbt,),
        in_specs=[
            pl.BlockSpec((C * L, TB), lambda j: (0, j)),
            pl.BlockSpec(wcf.shape, lambda j: (0, 0, 0)),
            pl.BlockSpec((1, _OC), lambda j: (0, 0)),
            pl.BlockSpec((_HID, L * _OC), lambda j: (0, 0)),
            pl.BlockSpec((1, _HID), lambda j: (0, 0)),
        ],
        out_specs=pl.BlockSpec((_HID, TB), lambda j: (0, j)),
        scratch_shapes=[
            pltpu.VMEM(((_L + 1) * _OC, _C * _L), jnp.bfloat16),
            pltpu.VMEM((_L * _OC, 1), f32),
            pltpu.VMEM((_HID, _L * _OC), jnp.bfloat16),
            pltpu.VMEM((_HID, 1), f32),
        ],
        compiler_params=pltpu.CompilerParams(
            dimension_semantics=("arbitrary",),
            vmem_limit_bytes=64 << 20),
    )(xt, wcf, bcf, wlf, blf)
    return jnp.transpose(out[:, :B])
